# Initial kernel scaffold; baseline (speedup 1.0000x reference)
import jax
import jax.numpy as jnp
from jax.experimental import pallas as pl


def kernel(edge_index, x, batch, pre_W, pre_b, W0, a_src0, a_dst0, b0, W1, a_src1, a_dst1, b1, W2, a_src2, a_dst2, b2):
    raise NotImplementedError("write your pallas kernel here")


# speedup vs baseline: 1.0884x; 1.0884x over previous
import functools

import jax
import jax.numpy as jnp
from jax import lax
from jax.experimental import pallas as pl
from jax.experimental.pallas import tpu as pltpu

NG = 64
NPER = 128
N = NG * NPER
E = NG * (NPER * (NPER - 1) // 2)
ROWS = 512
GRID = N // ROWS


def _st0_body(x_ref, preW_ref, preb_ref, W_ref, am_ref, dm_ref,
              h_ref, as_ref, ad_ref):
    y = jnp.dot(x_ref[...], preW_ref[...], preferred_element_type=jnp.float32)
    y = y + preb_ref[...][None, :]
    h = jnp.dot(y, W_ref[...], preferred_element_type=jnp.float32)
    h_ref[...] = h
    as_ref[...] = jnp.dot(h, am_ref[...], preferred_element_type=jnp.float32)
    ad_ref[...] = jnp.dot(h, dm_ref[...], preferred_element_type=jnp.float32)


def _stage0(x2, pre_W, pre_b, W0, am, dm, H):
    hc = W0.shape[1]
    return pl.pallas_call(
        _st0_body,
        grid=(GRID,),
        in_specs=[
            pl.BlockSpec((ROWS, x2.shape[1]), lambda i: (i, 0)),
            pl.BlockSpec(pre_W.shape, lambda i: (0, 0)),
            pl.BlockSpec(pre_b.shape, lambda i: (0,)),
            pl.BlockSpec(W0.shape, lambda i: (0, 0)),
            pl.BlockSpec(am.shape, lambda i: (0, 0)),
            pl.BlockSpec(dm.shape, lambda i: (0, 0)),
        ],
        out_specs=[
            pl.BlockSpec((ROWS, hc), lambda i: (i, 0)),
            pl.BlockSpec((ROWS, H), lambda i: (i, 0)),
            pl.BlockSpec((ROWS, H), lambda i: (i, 0)),
        ],
        out_shape=[
            jax.ShapeDtypeStruct((N, hc), jnp.float32),
            jax.ShapeDtypeStruct((N, H), jnp.float32),
            jax.ShapeDtypeStruct((N, H), jnp.float32),
        ],
    )(x2, pre_W, pre_b, W0, am, dm)


def _stt_body(p0_ref, p1_ref, b_ref, W_ref, am_ref, dm_ref,
              h_ref, as_ref, ad_ref):
    y = p0_ref[...] + p1_ref[...] + b_ref[...][None, :]
    y = jnp.maximum(y, 0.0)
    h = jnp.dot(y, W_ref[...], preferred_element_type=jnp.float32)
    h_ref[...] = h
    as_ref[...] = jnp.dot(h, am_ref[...], preferred_element_type=jnp.float32)
    ad_ref[...] = jnp.dot(h, dm_ref[...], preferred_element_type=jnp.float32)


def _staget(p0, p1, b, W, am, dm, H):
    hcin = p0.shape[1]
    hc = W.shape[1]
    return pl.pallas_call(
        _stt_body,
        grid=(GRID,),
        in_specs=[
            pl.BlockSpec((ROWS, hcin), lambda i: (i, 0)),
            pl.BlockSpec((ROWS, hcin), lambda i: (i, 0)),
            pl.BlockSpec(b.shape, lambda i: (0,)),
            pl.BlockSpec(W.shape, lambda i: (0, 0)),
            pl.BlockSpec(am.shape, lambda i: (0, 0)),
            pl.BlockSpec(dm.shape, lambda i: (0, 0)),
        ],
        out_specs=[
            pl.BlockSpec((ROWS, hc), lambda i: (i, 0)),
            pl.BlockSpec((ROWS, H), lambda i: (i, 0)),
            pl.BlockSpec((ROWS, H), lambda i: (i, 0)),
        ],
        out_shape=[
            jax.ShapeDtypeStruct((N, hc), jnp.float32),
            jax.ShapeDtypeStruct((N, H), jnp.float32),
            jax.ShapeDtypeStruct((N, H), jnp.float32),
        ],
    )(p0, p1, b, W, am, dm)


def _pool_body(p0_ref, p1_ref, b_ref, out_ref):
    y = p0_ref[...] + p1_ref[...] + b_ref[...][None, :]
    y = jnp.maximum(y, 0.0)
    c = y.shape[-1]
    out_ref[...] = jnp.mean(y.reshape(8, NPER, c), axis=1)


def _pool(p0, p1, b2):
    c = p0.shape[1]
    return pl.pallas_call(
        _pool_body,
        grid=(NG // 8,),
        in_specs=[
            pl.BlockSpec((8 * NPER, c), lambda i: (i, 0)),
            pl.BlockSpec((8 * NPER, c), lambda i: (i, 0)),
            pl.BlockSpec(b2.shape, lambda i: (0,)),
        ],
        out_specs=pl.BlockSpec((8, c), lambda i: (i, 0)),
        out_shape=jax.ShapeDtypeStruct((NG, c), jnp.float32),
    )(p0, p1, b2)


def _attn_mats(a_src, a_dst, H, C):
    eye = jnp.eye(H, dtype=jnp.float32)
    am = (a_src.reshape(H, C)[:, :, None] * eye[:, None, :]).reshape(H * C, H)
    dm = (a_dst.reshape(H, C)[:, :, None] * eye[:, None, :]).reshape(H * C, H)
    return am, dm


def _edge_aggregate(src, dst, h, as_, ad_, H, C):
    alpha = as_[src] + ad_[dst]
    alpha = jnp.where(alpha > 0, alpha, 0.2 * alpha)
    e = jnp.exp(alpha)
    s = jax.ops.segment_sum(e, dst, num_segments=N)
    w = e / (s[dst] + 1e-16)
    msg = h.reshape(N, H, C)[src] * w[:, :, None]
    out = jax.ops.segment_sum(msg, dst, num_segments=N)
    return out.reshape(N, H * C)


def kernel(edge_index, x, batch, pre_W, pre_b, W0, a_src0, a_dst0, b0,
           W1, a_src1, a_dst1, b1, W2, a_src2, a_dst2, b2):
    src = edge_index[0]
    dst = edge_index[1]
    x2 = x.reshape(-1, x.shape[-1])

    am0, dm0 = _attn_mats(a_src0, a_dst0, 4, 32)
    am1, dm1 = _attn_mats(a_src1, a_dst1, 4, 32)
    am2, dm2 = _attn_mats(a_src2, a_dst2, 1, 64)

    h0, as0, ad0 = _stage0(x2, pre_W, pre_b, W0, am0, dm0, 4)
    agg0 = _edge_aggregate(src, dst, h0, as0, ad0, 4, 32)
    z = jnp.zeros_like(agg0)

    h1, as1, ad1 = _staget(agg0, z, b0, W1, am1, dm1, 4)
    agg1 = _edge_aggregate(src, dst, h1, as1, ad1, 4, 32)

    h2, as2, ad2 = _staget(agg1, z, b1, W2, am2, dm2, 1)
    agg2 = _edge_aggregate(src, dst, h2, as2, ad2, 1, 64)

    return _pool(agg2, jnp.zeros_like(agg2), b2)


# speedup vs baseline: 18.9763x; 17.4344x over previous
import functools

import jax
import jax.numpy as jnp
from jax import lax
from jax.experimental import pallas as pl
from jax.experimental.pallas import tpu as pltpu
from jax.experimental.pallas import tpu_sc as plsc

NG = 64
NPER = 128
N = NG * NPER
E = NG * (NPER * (NPER - 1) // 2)
ROWS = 512
GRID = N // ROWS

NC, NS = 2, 16
NW = NC * NS
EW = E // NW
K = 128
NCHUNK = EW // K


def _st0_body(x_ref, preW_ref, preb_ref, W_ref, am_ref, dm_ref,
              h_ref, as_ref, ad_ref):
    y = jnp.dot(x_ref[...], preW_ref[...], preferred_element_type=jnp.float32)
    y = y + preb_ref[...][None, :]
    h = jnp.dot(y, W_ref[...], preferred_element_type=jnp.float32)
    h_ref[...] = h
    as_ref[...] = jnp.dot(h, am_ref[...], preferred_element_type=jnp.float32)
    ad_ref[...] = jnp.dot(h, dm_ref[...], preferred_element_type=jnp.float32)


def _stage0(x2, pre_W, pre_b, W0, am, dm, H):
    hc = W0.shape[1]
    return pl.pallas_call(
        _st0_body,
        grid=(GRID,),
        in_specs=[
            pl.BlockSpec((ROWS, x2.shape[1]), lambda i: (i, 0)),
            pl.BlockSpec(pre_W.shape, lambda i: (0, 0)),
            pl.BlockSpec(pre_b.shape, lambda i: (0,)),
            pl.BlockSpec(W0.shape, lambda i: (0, 0)),
            pl.BlockSpec(am.shape, lambda i: (0, 0)),
            pl.BlockSpec(dm.shape, lambda i: (0, 0)),
        ],
        out_specs=[
            pl.BlockSpec((ROWS, hc), lambda i: (i, 0)),
            pl.BlockSpec((ROWS, H), lambda i: (i, 0)),
            pl.BlockSpec((ROWS, H), lambda i: (i, 0)),
        ],
        out_shape=[
            jax.ShapeDtypeStruct((N, hc), jnp.float32),
            jax.ShapeDtypeStruct((N, H), jnp.float32),
            jax.ShapeDtypeStruct((N, H), jnp.float32),
        ],
    )(x2, pre_W, pre_b, W0, am, dm)


def _stt_body(p0_ref, p1_ref, b_ref, W_ref, am_ref, dm_ref,
              h_ref, as_ref, ad_ref):
    y = p0_ref[...] + p1_ref[...] + b_ref[...][None, :]
    y = jnp.maximum(y, 0.0)
    h = jnp.dot(y, W_ref[...], preferred_element_type=jnp.float32)
    h_ref[...] = h
    as_ref[...] = jnp.dot(h, am_ref[...], preferred_element_type=jnp.float32)
    ad_ref[...] = jnp.dot(h, dm_ref[...], preferred_element_type=jnp.float32)


def _staget(p0, p1, b, W, am, dm, H):
    hcin = p0.shape[1]
    hc = W.shape[1]
    return pl.pallas_call(
        _stt_body,
        grid=(GRID,),
        in_specs=[
            pl.BlockSpec((ROWS, hcin), lambda i: (i, 0)),
            pl.BlockSpec((ROWS, hcin), lambda i: (i, 0)),
            pl.BlockSpec(b.shape, lambda i: (0,)),
            pl.BlockSpec(W.shape, lambda i: (0, 0)),
            pl.BlockSpec(am.shape, lambda i: (0, 0)),
            pl.BlockSpec(dm.shape, lambda i: (0, 0)),
        ],
        out_specs=[
            pl.BlockSpec((ROWS, hc), lambda i: (i, 0)),
            pl.BlockSpec((ROWS, H), lambda i: (i, 0)),
            pl.BlockSpec((ROWS, H), lambda i: (i, 0)),
        ],
        out_shape=[
            jax.ShapeDtypeStruct((N, hc), jnp.float32),
            jax.ShapeDtypeStruct((N, H), jnp.float32),
            jax.ShapeDtypeStruct((N, H), jnp.float32),
        ],
    )(p0, p1, b, W, am, dm)


def _pool_body(p0_ref, p1_ref, b_ref, out_ref):
    c = b_ref.shape[0]
    y = p0_ref[...][:, :c] + p1_ref[...][:, :c] + b_ref[...][None, :]
    y = jnp.maximum(y, 0.0)
    out_ref[...] = jnp.mean(y.reshape(8, NPER, c), axis=1)


def _pool(p0, p1, b2):
    cin = p0.shape[1]
    c = b2.shape[0]
    return pl.pallas_call(
        _pool_body,
        grid=(NG // 8,),
        in_specs=[
            pl.BlockSpec((8 * NPER, cin), lambda i: (i, 0)),
            pl.BlockSpec((8 * NPER, cin), lambda i: (i, 0)),
            pl.BlockSpec(b2.shape, lambda i: (0,)),
        ],
        out_specs=pl.BlockSpec((8, c), lambda i: (i, 0)),
        out_shape=jax.ShapeDtypeStruct((NG, c), jnp.float32),
    )(p0, p1, b2)


def _attn_mats(a_src, a_dst, H, C):
    eye = jnp.eye(H, dtype=jnp.float32)
    am = (a_src.reshape(H, C)[:, :, None] * eye[:, None, :]).reshape(H * C, H)
    dm = (a_dst.reshape(H, C)[:, :, None] * eye[:, None, :]).reshape(H * C, H)
    return am, dm


def _make_passA(H):
    NH = N * H
    NR = NH // 128
    NRB = (NR + 127) // 128
    RB = min(NR, 128)
    TR = NR // NS
    mesh = plsc.VectorSubcoreMesh(core_axis_name="c", subcore_axis_name="s")

    @functools.partial(
        pl.kernel,
        out_type=[
            jax.ShapeDtypeStruct((H, E), jnp.float32),
            jax.ShapeDtypeStruct((2, NR, 128), jnp.float32),
        ],
        mesh=mesh,
        compiler_params=pltpu.CompilerParams(needs_layout_passes=False),
        scratch_types=[
            pltpu.VMEM((NH,), jnp.float32),
            pltpu.VMEM((NH,), jnp.float32),
            pltpu.VMEM((NR, 128), jnp.float32),
            pltpu.VMEM((K,), jnp.int32),
            pltpu.VMEM((K,), jnp.int32),
            pltpu.VMEM((H, K), jnp.float32),
            pltpu.VMEM((NRB, RB), jnp.int32),
            pltpu.VMEM((TR, 128), jnp.float32),
            pltpu.VMEM_SHARED((NR, 128), jnp.float32),
        ],
    )
    def passA(src_hbm, dst_hbm, as_hbm, ad_hbm, pT_hbm, s_hbm,
              as_t, ad_t, s_t, src_c, dst_c, p_c, idxrow, zbuf, shared_s):
        cid = lax.axis_index("c")
        sid = lax.axis_index("s")
        wid = cid * NS + sid
        pltpu.sync_copy(as_hbm, as_t)
        pltpu.sync_copy(ad_hbm, ad_t)
        zv = jnp.zeros((16,), jnp.float32)
        iota16 = lax.iota(jnp.int32, 16)

        for b in range(NRB):
            for j in range(RB // 16):
                idxrow[b, pl.ds(j * 16, 16)] = iota16 + (b * RB + j * 16)
        for i in range(TR):
            for j in range(8):
                zbuf[i, pl.ds(j * 16, 16)] = zv

        @pl.loop(0, NR)
        def _(r):
            for j in range(8):
                s_t[r, pl.ds(j * 16, 16)] = zv

        pltpu.sync_copy(zbuf, shared_s.at[pl.ds(sid * TR, TR)])
        plsc.subcore_barrier()

        ebase = wid * EW

        @pl.loop(0, NCHUNK)
        def _(ci):
            base = ebase + ci * K
            pltpu.sync_copy(src_hbm.at[pl.ds(base, K)], src_c)
            pltpu.sync_copy(dst_hbm.at[pl.ds(base, K)], dst_c)

            @pl.loop(0, K // 16)
            def _(g):
                s16 = src_c[pl.ds(g * 16, 16)] * H
                d16 = dst_c[pl.ds(g * 16, 16)] * H
                for h in range(H):
                    a = (plsc.load_gather(as_t, [s16 + h])
                         + plsc.load_gather(ad_t, [d16 + h]))
                    a = jnp.where(a > 0, a, 0.2 * a)
                    p = jnp.exp(a)
                    p_c[h, pl.ds(g * 16, 16)] = p
                    flat = d16 + h
                    plsc.addupdate_scatter(
                        s_t, [lax.shift_right_logical(flat, 7),
                              lax.bitwise_and(flat, 127)], p)

            for h in range(H):
                pltpu.sync_copy(p_c.at[h], pT_hbm.at[h, pl.ds(base, K)])

        for b in range(NRB):
            pltpu.sync_copy(s_t.at[pl.ds(b * RB, RB)],
                            shared_s.at[idxrow.at[b]], add=True)
        plsc.subcore_barrier()
        pltpu.sync_copy(shared_s.at[pl.ds(sid * TR, TR)],
                        s_hbm.at[cid, pl.ds(sid * TR, TR)])

    return passA


def _make_passB(H, C, CW):
    NH = N * H
    NR = NH // 128
    RSL = N // NS
    mesh = plsc.VectorSubcoreMesh(core_axis_name="c", subcore_axis_name="s")

    @functools.partial(
        pl.kernel,
        out_type=jax.ShapeDtypeStruct((2, N, CW), jnp.float32),
        mesh=mesh,
        compiler_params=pltpu.CompilerParams(needs_layout_passes=False),
        scratch_types=[
            pltpu.VMEM((NH,), jnp.float32),
            pltpu.VMEM((16, 128), jnp.float32),
            pltpu.VMEM((16, 128), jnp.float32),
            pltpu.VMEM((K,), jnp.int32),
            pltpu.VMEM((K,), jnp.int32),
            pltpu.VMEM((H, K), jnp.float32),
            pltpu.VMEM((K, CW), jnp.float32),
            pltpu.VMEM_SHARED((N, CW), jnp.float32),
            pltpu.SemaphoreType.DMA,
        ],
    )
    def passB(src_hbm, dst_hbm, pT_hbm, s_hbm, h_hbm, out_hbm,
              r_tab, sbuf0, sbuf1, src_c, dst_c, p_c, rows,
              shared_out, sem):
        cid = lax.axis_index("c")
        sid = lax.axis_index("s")
        wid = cid * NS + sid

        @pl.loop(0, NR // 16)
        def _(b):
            pltpu.sync_copy(s_hbm.at[0, pl.ds(b * 16, 16)], sbuf0)
            pltpu.sync_copy(s_hbm.at[1, pl.ds(b * 16, 16)], sbuf1)

            @pl.loop(0, 16)
            def _(i):
                for j in range(8):
                    s = sbuf0[i, pl.ds(j * 16, 16)] + sbuf1[i, pl.ds(j * 16, 16)]
                    r_tab[pl.ds((b * 16 + i) * 128 + j * 16, 16)] = (
                        1.0 / (s + 1e-16))

        zv = jnp.zeros((16,), jnp.float32)

        @pl.loop(0, 16)
        def _(i):
            for j in range(CW // 16):
                rows[i, pl.ds(j * 16, 16)] = zv

        @pl.loop(0, RSL // 16)
        def _(j):
            pltpu.sync_copy(rows.at[pl.ds(0, 16)],
                            shared_out.at[pl.ds(sid * RSL + j * 16, 16)])

        plsc.subcore_barrier()

        iota16 = lax.iota(jnp.int32, 16)
        ebase = wid * EW

        @pl.loop(0, NCHUNK)
        def _(ci):
            base = ebase + ci * K
            pltpu.sync_copy(src_hbm.at[pl.ds(base, K)], src_c)
            pltpu.sync_copy(dst_hbm.at[pl.ds(base, K)], dst_c)
            for h in range(H):
                pltpu.sync_copy(pT_hbm.at[h, pl.ds(base, K)], p_c.at[h])
            pltpu.async_copy(h_hbm.at[src_c], rows, sem).wait()

            @pl.loop(0, K // 16)
            def _(g):
                d16 = dst_c[pl.ds(g * 16, 16)] * H
                rowidx = g * 16 + iota16
                for h in range(H):
                    r16 = plsc.load_gather(r_tab, [d16 + h])
                    w = p_c[h, pl.ds(g * 16, 16)] * r16
                    for f in range(C):
                        cv = jnp.full((16,), h * C + f, jnp.int32)
                        col = plsc.load_gather(rows, [rowidx, cv])
                        plsc.store_scatter(rows, [rowidx, cv], col * w)

            pltpu.sync_copy(rows, shared_out.at[dst_c], add=True)

        plsc.subcore_barrier()

        @pl.loop(0, RSL // 64)
        def _(j):
            r0 = sid * RSL + j * 64
            pltpu.sync_copy(shared_out.at[pl.ds(r0, 64)],
                            out_hbm.at[cid, pl.ds(r0, 64)])

    return passB


_passA4 = _make_passA(4)
_passA1 = _make_passA(1)
_passB4 = _make_passB(4, 32, 128)
_passB1 = _make_passB(1, 64, 128)


def _edge_aggregate(src, dst, h, as_, ad_, H):
    passA = _passA4 if H == 4 else _passA1
    passB = _passB4 if H == 4 else _passB1
    pT, s_parts = passA(src, dst, as_.reshape(-1), ad_.reshape(-1))
    out_parts = passB(src, dst, pT, s_parts, h)
    return out_parts[0], out_parts[1]


def kernel(edge_index, x, batch, pre_W, pre_b, W0, a_src0, a_dst0, b0,
           W1, a_src1, a_dst1, b1, W2, a_src2, a_dst2, b2):
    src = edge_index[0]
    dst = edge_index[1]
    x2 = x.reshape(-1, x.shape[-1])

    am0, dm0 = _attn_mats(a_src0, a_dst0, 4, 32)
    am1, dm1 = _attn_mats(a_src1, a_dst1, 4, 32)
    am2, dm2 = _attn_mats(a_src2, a_dst2, 1, 64)
    W2p = jnp.concatenate([W2, jnp.zeros((W2.shape[0], 64), jnp.float32)], 1)
    am2p = jnp.concatenate([am2, jnp.zeros((64, 1), jnp.float32)], 0)
    dm2p = jnp.concatenate([dm2, jnp.zeros((64, 1), jnp.float32)], 0)

    h0, as0, ad0 = _stage0(x2, pre_W, pre_b, W0, am0, dm0, 4)
    a0p0, a0p1 = _edge_aggregate(src, dst, h0, as0, ad0, 4)

    h1, as1, ad1 = _staget(a0p0, a0p1, b0, W1, am1, dm1, 4)
    a1p0, a1p1 = _edge_aggregate(src, dst, h1, as1, ad1, 4)

    h2, as2, ad2 = _staget(a1p0, a1p1, b1, W2p, am2p, dm2p, 1)
    a2p0, a2p1 = _edge_aggregate(src, dst, h2, as2, ad2, 1)

    return _pool(a2p0, a2p1, b2)


# speedup vs baseline: 86.5629x; 4.5616x over previous
import functools

import jax
import jax.numpy as jnp
from jax import lax
from jax.experimental import pallas as pl
from jax.experimental.pallas import tpu as pltpu
from jax.experimental.pallas import tpu_sc as plsc

NG = 64
NPER = 128
N = NG * NPER
E = NG * (NPER * (NPER - 1) // 2)
ROWS = 512
GRID = N // ROWS

NC, NS = 2, 16
NW = NC * NS
EW = E // NW
K = 128
NCHUNK = EW // K


def _st0_body(x_ref, preW_ref, preb_ref, W_ref, am_ref, dm_ref,
              h_ref, as_ref, ad_ref):
    y = jnp.dot(x_ref[...], preW_ref[...], preferred_element_type=jnp.float32)
    y = y + preb_ref[...][None, :]
    h = jnp.dot(y, W_ref[...], preferred_element_type=jnp.float32)
    h_ref[...] = h
    as_ref[...] = jnp.dot(h, am_ref[...], preferred_element_type=jnp.float32)
    ad_ref[...] = jnp.dot(h, dm_ref[...], preferred_element_type=jnp.float32)


def _stage0(x2, pre_W, pre_b, W0, am, dm, H):
    hc = W0.shape[1]
    return pl.pallas_call(
        _st0_body,
        grid=(GRID,),
        in_specs=[
            pl.BlockSpec((ROWS, x2.shape[1]), lambda i: (i, 0)),
            pl.BlockSpec(pre_W.shape, lambda i: (0, 0)),
            pl.BlockSpec(pre_b.shape, lambda i: (0,)),
            pl.BlockSpec(W0.shape, lambda i: (0, 0)),
            pl.BlockSpec(am.shape, lambda i: (0, 0)),
            pl.BlockSpec(dm.shape, lambda i: (0, 0)),
        ],
        out_specs=[
            pl.BlockSpec((ROWS, hc), lambda i: (i, 0)),
            pl.BlockSpec((ROWS, H), lambda i: (i, 0)),
            pl.BlockSpec((ROWS, H), lambda i: (i, 0)),
        ],
        out_shape=[
            jax.ShapeDtypeStruct((N, hc), jnp.float32),
            jax.ShapeDtypeStruct((N, H), jnp.float32),
            jax.ShapeDtypeStruct((N, H), jnp.float32),
        ],
    )(x2, pre_W, pre_b, W0, am, dm)


def _stt_body(Hin, p0_ref, p1_ref, s_ref, em_ref, b_ref, W_ref, am_ref,
              dm_ref, h_ref, as_ref, ad_ref):
    s = s_ref[0] + s_ref[1]
    rr = 1.0 / (s + 1e-16)
    rexp = lax.dot_general(rr, em_ref[...], (((0,), (0,)), ((), ())),
                           preferred_element_type=jnp.float32)
    agg = (p0_ref[...] + p1_ref[...])[:, :rexp.shape[1]] * rexp
    y = jnp.maximum(agg + b_ref[...][None, :], 0.0)
    h = jnp.dot(y, W_ref[...], preferred_element_type=jnp.float32)
    h_ref[...] = h
    as_ref[...] = jnp.dot(h, am_ref[...], preferred_element_type=jnp.float32)
    ad_ref[...] = jnp.dot(h, dm_ref[...], preferred_element_type=jnp.float32)


def _staget(p0, p1, s_hm, b, W, am, dm, Hin, H):
    hcin = p0.shape[1]
    hc = W.shape[1]
    em = jnp.repeat(jnp.eye(Hin, dtype=jnp.float32), hcin // Hin, axis=1)
    return pl.pallas_call(
        functools.partial(_stt_body, Hin),
        grid=(GRID,),
        in_specs=[
            pl.BlockSpec((ROWS, hcin), lambda i: (i, 0)),
            pl.BlockSpec((ROWS, hcin), lambda i: (i, 0)),
            pl.BlockSpec((2, Hin, ROWS), lambda i: (0, 0, i)),
            pl.BlockSpec(em.shape, lambda i: (0, 0)),
            pl.BlockSpec(b.shape, lambda i: (0,)),
            pl.BlockSpec(W.shape, lambda i: (0, 0)),
            pl.BlockSpec(am.shape, lambda i: (0, 0)),
            pl.BlockSpec(dm.shape, lambda i: (0, 0)),
        ],
        out_specs=[
            pl.BlockSpec((ROWS, hc), lambda i: (i, 0)),
            pl.BlockSpec((ROWS, H), lambda i: (i, 0)),
            pl.BlockSpec((ROWS, H), lambda i: (i, 0)),
        ],
        out_shape=[
            jax.ShapeDtypeStruct((N, hc), jnp.float32),
            jax.ShapeDtypeStruct((N, H), jnp.float32),
            jax.ShapeDtypeStruct((N, H), jnp.float32),
        ],
    )(p0, p1, s_hm, em, b, W, am, dm)


def _pool_body(p0_ref, p1_ref, s_ref, em_ref, b_ref, out_ref):
    c = b_ref.shape[0]
    s = s_ref[0] + s_ref[1]
    rr = 1.0 / (s + 1e-16)
    rexp = lax.dot_general(rr, em_ref[...], (((0,), (0,)), ((), ())),
                           preferred_element_type=jnp.float32)
    y = (p0_ref[...][:, :c] + p1_ref[...][:, :c]) * rexp
    y = jnp.maximum(y + b_ref[...][None, :], 0.0)
    out_ref[...] = jnp.mean(y.reshape(8, NPER, c), axis=1)


def _pool(p0, p1, s_hm, b2):
    cin = p0.shape[1]
    c = b2.shape[0]
    em = jnp.ones((1, c), jnp.float32)
    return pl.pallas_call(
        _pool_body,
        grid=(NG // 8,),
        in_specs=[
            pl.BlockSpec((8 * NPER, cin), lambda i: (i, 0)),
            pl.BlockSpec((8 * NPER, cin), lambda i: (i, 0)),
            pl.BlockSpec((2, 1, 8 * NPER), lambda i: (0, 0, i)),
            pl.BlockSpec(em.shape, lambda i: (0, 0)),
            pl.BlockSpec(b2.shape, lambda i: (0,)),
        ],
        out_specs=pl.BlockSpec((8, c), lambda i: (i, 0)),
        out_shape=jax.ShapeDtypeStruct((NG, c), jnp.float32),
    )(p0, p1, s_hm, em, b2)


def _attn_mats(a_src, a_dst, H, C):
    eye = jnp.eye(H, dtype=jnp.float32)
    am = (a_src.reshape(H, C)[:, :, None] * eye[:, None, :]).reshape(H * C, H)
    dm = (a_dst.reshape(H, C)[:, :, None] * eye[:, None, :]).reshape(H * C, H)
    return am, dm


def _make_passA(H):
    NH = N * H
    NR = NH // 128
    NRB = (NR + 127) // 128
    RB = min(NR, 128)
    TR = NR // NS
    mesh = plsc.VectorSubcoreMesh(core_axis_name="c", subcore_axis_name="s")

    KA = 2032
    NCHA = EW // KA

    @functools.partial(
        pl.kernel,
        out_type=[
            jax.ShapeDtypeStruct((E * H,), jnp.float32),
            jax.ShapeDtypeStruct((2, NR, 128), jnp.float32),
        ],
        mesh=mesh,
        compiler_params=pltpu.CompilerParams(needs_layout_passes=False),
        scratch_types=[
            pltpu.VMEM((NH,), jnp.float32),
            pltpu.VMEM((NH,), jnp.float32),
            pltpu.VMEM((NR, 128), jnp.float32),
            pltpu.VMEM((KA,), jnp.int32),
            pltpu.VMEM((KA,), jnp.int32),
            pltpu.VMEM((KA * H,), jnp.float32),
            pltpu.VMEM((NRB, RB), jnp.int32),
            pltpu.VMEM((TR, 128), jnp.float32),
            pltpu.VMEM_SHARED((NR, 128), jnp.float32),
        ],
    )
    def passA(src_hbm, dst_hbm, as_hbm, ad_hbm, pE_hbm, s_hbm,
              as_t, ad_t, s_t, src_c, dst_c, p_c, idxrow, zbuf, shared_s):
        cid = lax.axis_index("c")
        sid = lax.axis_index("s")
        wid = cid * NS + sid
        pltpu.sync_copy(as_hbm, as_t)
        pltpu.sync_copy(ad_hbm, ad_t)
        zv = jnp.zeros((16,), jnp.float32)
        iota16 = lax.iota(jnp.int32, 16)

        for b in range(NRB):
            for j in range(RB // 16):
                idxrow[b, pl.ds(j * 16, 16)] = iota16 + (b * RB + j * 16)
        for i in range(TR):
            for j in range(8):
                zbuf[i, pl.ds(j * 16, 16)] = zv

        @pl.loop(0, NR)
        def _(r):
            for j in range(8):
                s_t[r, pl.ds(j * 16, 16)] = zv

        pltpu.sync_copy(zbuf, shared_s.at[pl.ds(sid * TR, TR)])
        plsc.subcore_barrier()

        ebase = wid * EW
        iota16 = lax.iota(jnp.int32, 16)

        @pl.loop(0, NCHA)
        def _(ci):
            base = ebase + ci * KA
            pltpu.sync_copy(src_hbm.at[pl.ds(base, KA)], src_c)
            pltpu.sync_copy(dst_hbm.at[pl.ds(base, KA)], dst_c)

            @pl.loop(0, KA // 16)
            def _(g):
                s16 = src_c[pl.ds(g * 16, 16)] * H
                d16 = dst_c[pl.ds(g * 16, 16)]
                d16H = d16 * H
                pidx = (g * 16 + iota16) * H
                for h in range(H):
                    a = (plsc.load_gather(as_t, [s16 + h])
                         + plsc.load_gather(ad_t, [d16H + h]))
                    a = jnp.where(a > 0, a, 0.2 * a)
                    p = jnp.exp(a)
                    plsc.store_scatter(p_c, [pidx + h], p)
                    flat = d16 + h * N
                    plsc.addupdate_scatter(
                        s_t, [lax.shift_right_logical(flat, 7),
                              lax.bitwise_and(flat, 127)], p)

            pltpu.sync_copy(p_c, pE_hbm.at[pl.ds(base * H, KA * H)])

        for b in range(NRB):
            pltpu.sync_copy(s_t.at[pl.ds(b * RB, RB)],
                            shared_s.at[idxrow.at[b]], add=True)
        plsc.subcore_barrier()
        pltpu.sync_copy(shared_s.at[pl.ds(sid * TR, TR)],
                        s_hbm.at[cid, pl.ds(sid * TR, TR)])

    return passA


def _make_passB(H, C, CW):
    NH = N * H
    RSL = N // NS
    mesh = plsc.VectorSubcoreMesh(core_axis_name="c", subcore_axis_name="s")

    @functools.partial(
        pl.kernel,
        out_type=jax.ShapeDtypeStruct((2, N, CW), jnp.float32),
        mesh=mesh,
        compiler_params=pltpu.CompilerParams(needs_layout_passes=False),
        scratch_types=[
            pltpu.VMEM((K,), jnp.int32),
            pltpu.VMEM((K,), jnp.int32),
            pltpu.VMEM((K * H,), jnp.float32),
            pltpu.VMEM((K, CW), jnp.float32),
            pltpu.VMEM_SHARED((N, CW), jnp.float32),
            pltpu.SemaphoreType.DMA,
        ],
    )
    def passB(src_hbm, dst_hbm, h_hbm, pE_hbm, out_hbm,
              src_c, dst_c, p_c, rows,
              shared_out, sem):
        cid = lax.axis_index("c")
        sid = lax.axis_index("s")
        wid = cid * NS + sid

        zv = jnp.zeros((16,), jnp.float32)

        @pl.loop(0, 16)
        def _(i):
            for j in range(CW // 16):
                rows[i, pl.ds(j * 16, 16)] = zv

        @pl.loop(0, RSL // 16)
        def _(j):
            pltpu.sync_copy(rows.at[pl.ds(0, 16)],
                            shared_out.at[pl.ds(sid * RSL + j * 16, 16)])

        plsc.subcore_barrier()

        ebase = wid * EW

        @pl.loop(0, NCHUNK)
        def _(ci):
            base = ebase + ci * K
            pltpu.sync_copy(src_hbm.at[pl.ds(base, K)], src_c)
            pltpu.sync_copy(pE_hbm.at[pl.ds(base * H, K * H)], p_c)
            pltpu.sync_copy(dst_hbm.at[pl.ds(base, K)], dst_c)
            pltpu.async_copy(h_hbm.at[src_c], rows, sem).wait()

            @pl.loop(0, K // 16)
            def _(g):
                pv = [p_c[pl.ds(g * 16 * H + j * 16, 16)]
                      for j in range(H)]
                for l in range(16):
                    e = g * 16 + l
                    for h in range(H):
                        q = l * H + h
                        w = jnp.full((16,), pv[q // 16][q % 16], jnp.float32)
                        for j in range(C // 16):
                            col = h * C + j * 16
                            rows[e, pl.ds(col, 16)] = (
                                rows[e, pl.ds(col, 16)] * w)

            pltpu.sync_copy(rows, shared_out.at[dst_c], add=True)

        plsc.subcore_barrier()

        @pl.loop(0, RSL // 64)
        def _(j):
            r0 = sid * RSL + j * 64
            pltpu.sync_copy(shared_out.at[pl.ds(r0, 64)],
                            out_hbm.at[cid, pl.ds(r0, 64)])

    return passB


_passA4 = _make_passA(4)
_passA1 = _make_passA(1)
_passB4 = _make_passB(4, 32, 128)
_passB1 = _make_passB(1, 64, 128)


def _edge_aggregate(src, dst, h, as_, ad_, H):
    passA = _passA4 if H == 4 else _passA1
    passB = _passB4 if H == 4 else _passB1
    asf = as_.reshape(-1)
    adf = ad_.reshape(-1)
    pE, s_parts = passA(src, dst, asf, adf)
    out_parts = passB(src, dst, h, pE)
    return out_parts, s_parts.reshape(2, H, N)


def kernel(edge_index, x, batch, pre_W, pre_b, W0, a_src0, a_dst0, b0,
           W1, a_src1, a_dst1, b1, W2, a_src2, a_dst2, b2):
    src = edge_index[0]
    dst = edge_index[1]
    x2 = x.reshape(-1, x.shape[-1])

    am0, dm0 = _attn_mats(a_src0, a_dst0, 4, 32)
    am1, dm1 = _attn_mats(a_src1, a_dst1, 4, 32)
    am2, dm2 = _attn_mats(a_src2, a_dst2, 1, 64)
    W2p = jnp.concatenate([W2, jnp.zeros((W2.shape[0], 64), jnp.float32)], 1)
    am2p = jnp.concatenate([am2, jnp.zeros((64, 1), jnp.float32)], 0)
    dm2p = jnp.concatenate([dm2, jnp.zeros((64, 1), jnp.float32)], 0)

    h0, as0, ad0 = _stage0(x2, pre_W, pre_b, W0, am0, dm0, 4)
    o0, s0 = _edge_aggregate(src, dst, h0, as0, ad0, 4)

    h1, as1, ad1 = _staget(o0[0], o0[1], s0, b0, W1, am1, dm1, 4, 4)
    o1, s1 = _edge_aggregate(src, dst, h1, as1, ad1, 4)

    h2, as2, ad2 = _staget(o1[0], o1[1], s1, b1, W2p, am2p, dm2p, 4, 1)
    o2, s2 = _edge_aggregate(src, dst, h2, as2, ad2, 1)

    return _pool(o2[0], o2[1], s2, b2)
